# baseline (speedup 1.0000x reference)
import functools

import jax
import jax.numpy as jnp
from jax import lax
from jax.experimental import pallas as pl
from jax.experimental.pallas import tpu as pltpu
from jax.experimental.pallas import tpu_sc as plsc

T, D, E, F = 2048, 2048, 8, 4096
K = 2
TK = T * K
M = 1024
NB = TK // M + E
P = NB * M
FB = 512
NF = F // FB

RB = 512
NC, NS = 2, 16
NW = NC * NS


def _router_body(x_ref, wr_ref, a1_ref, a2_ref, g1_ref, g2_ref, cs_ref):
    r = pl.program_id(0)
    x = x_ref[...]
    logits = jnp.dot(x, wr_ref[...], preferred_element_type=jnp.float32)
    iota = lax.broadcasted_iota(jnp.int32, (RB, E), 1)
    m1 = jnp.max(logits, axis=1, keepdims=True)
    is1 = logits == m1
    a1 = jnp.min(jnp.where(is1, iota, E), axis=1, keepdims=True)
    l2 = jnp.where(iota == a1, -jnp.inf, logits)
    m2 = jnp.max(l2, axis=1, keepdims=True)
    a2 = jnp.min(jnp.where(l2 == m2, iota, E), axis=1, keepdims=True)
    g1 = jax.nn.sigmoid(m1 - m2)
    a1_ref[...] = a1
    a2_ref[...] = a2
    g1_ref[...] = g1
    g2_ref[...] = 1.0 - g1
    z = jnp.exp(logits - m1)
    probs = z / jnp.sum(z, axis=1, keepdims=True)
    part = jnp.sum(probs, axis=0, keepdims=True)

    @pl.when(r == 0)
    def _():
        cs_ref[...] = part

    @pl.when(r != 0)
    def _():
        cs_ref[...] += part


def _router(flat, W_router):
    nr = T // RB
    outs = (
        jax.ShapeDtypeStruct((T, 1), jnp.int32),
        jax.ShapeDtypeStruct((T, 1), jnp.int32),
        jax.ShapeDtypeStruct((T, 1), jnp.float32),
        jax.ShapeDtypeStruct((T, 1), jnp.float32),
        jax.ShapeDtypeStruct((1, E), jnp.float32),
    )
    col = pl.BlockSpec((RB, 1), lambda r: (r, 0))
    return pl.pallas_call(
        _router_body,
        grid=(nr,),
        in_specs=[
            pl.BlockSpec((RB, D), lambda r: (r, 0)),
            pl.BlockSpec((D, E), lambda r: (0, 0)),
        ],
        out_specs=(col, col, col, col, pl.BlockSpec((1, E), lambda r: (0, 0))),
        out_shape=outs,
    )(flat, W_router)


DC = 16
DNCH = TK // NW // DC


def _dispatch_body(src_hbm, tok_hbm, pos_hbm, out_hbm,
                   tok_v, pos_v, bufs, gsem, ssem):
    wid = lax.axis_index("s") * NC + lax.axis_index("c")
    pltpu.sync_copy(tok_hbm.at[wid], tok_v)
    pltpu.sync_copy(pos_hbm.at[wid], pos_v)
    gathers = []
    scatters = []
    gathers.append(pltpu.async_copy(src_hbm.at[tok_v.at[0]], bufs.at[0], gsem))
    for c in range(DNCH):
        gathers[c].wait()
        if c + 1 < DNCH:
            if c >= 1:
                scatters[c - 1].wait()
            gathers.append(pltpu.async_copy(
                src_hbm.at[tok_v.at[c + 1]], bufs.at[(c + 1) % 2], gsem))
        scatters.append(pltpu.async_copy(
            bufs.at[c % 2], out_hbm.at[pos_v.at[c]], ssem))
    scatters[DNCH - 2].wait()
    scatters[DNCH - 1].wait()


def _dispatch(flat, tok_ids, pad_pos):
    mesh = plsc.VectorSubcoreMesh(core_axis_name="c", subcore_axis_name="s")
    k = pl.kernel(
        _dispatch_body,
        out_type=jax.ShapeDtypeStruct((P, D), jnp.float32),
        mesh=mesh,
        scratch_types=[
            pltpu.VMEM((DNCH, DC), jnp.int32),
            pltpu.VMEM((DNCH, DC), jnp.int32),
            pltpu.VMEM((2, DC, D), jnp.float32),
            pltpu.SemaphoreType.DMA,
            pltpu.SemaphoreType.DMA,
        ],
    )
    return k(flat, tok_ids.reshape(NW, DNCH, DC), pad_pos.reshape(NW, DNCH, DC))


def _ffn_body(jm_ref, fm_ref, em_ref, ns_ref,
              x_ref, g_ref, w1_ref, b1_ref, w2_ref, b2_ref, o_ref):
    g_id = pl.program_id(0)
    active = g_id < ns_ref[0]
    f = fm_ref[g_id]

    @pl.when(active)
    def _():
        x = x_ref[...].astype(jnp.bfloat16)
        w1 = w1_ref[0].astype(jnp.bfloat16)
        h = jnp.dot(x, w1, preferred_element_type=jnp.float32)
        h = jax.nn.gelu(h + b1_ref[0])
        w2 = w2_ref[0].astype(jnp.bfloat16)
        y = jnp.dot(h.astype(jnp.bfloat16), w2, preferred_element_type=jnp.float32)
        g = g_ref[0]
        val = y * g

        @pl.when(f == 0)
        def _():
            o_ref[...] = val + b2_ref[0] * g

        @pl.when(f != 0)
        def _():
            o_ref[...] += val


def _ffn(jmap, fmap, emap, nsteps, x_pad, gate_pad, W1, b1, W2, b2):
    gate3 = gate_pad.reshape(NB, M, 1)
    b1r = b1.reshape(E, 1, F)
    b2r = b2.reshape(E, 1, D)
    grid_spec = pltpu.PrefetchScalarGridSpec(
        num_scalar_prefetch=4,
        grid=(NB * NF,),
        in_specs=[
            pl.BlockSpec((M, D), lambda g, jm, fm, em, ns: (jm[g], 0)),
            pl.BlockSpec((1, M, 1), lambda g, jm, fm, em, ns: (jm[g], 0, 0)),
            pl.BlockSpec((1, D, FB), lambda g, jm, fm, em, ns: (em[g], 0, fm[g])),
            pl.BlockSpec((1, 1, FB), lambda g, jm, fm, em, ns: (em[g], 0, fm[g])),
            pl.BlockSpec((1, FB, D), lambda g, jm, fm, em, ns: (em[g], fm[g], 0)),
            pl.BlockSpec((1, 1, D), lambda g, jm, fm, em, ns: (em[g], 0, 0)),
        ],
        out_specs=pl.BlockSpec((M, D), lambda g, jm, fm, em, ns: (jm[g], 0)),
    )
    return pl.pallas_call(
        _ffn_body,
        grid_spec=grid_spec,
        out_shape=jax.ShapeDtypeStruct((P, D), jnp.float32),
    )(jmap, fmap, emap, nsteps, x_pad, gate3, W1, b1r, W2, b2r)


def _combine_body(y_hbm, pos_hbm, out_hbm, idx_v, rows_v, out_v, sem):
    wid = lax.axis_index("s") * NC + lax.axis_index("c")
    tok_per_w = T // NW
    TC_ = 16
    for c in range(tok_per_w // TC_):
        tbase = wid * tok_per_w + c * TC_
        pltpu.sync_copy(pos_hbm.at[pl.ds(tbase * K, TC_ * K)], idx_v)
        pltpu.async_copy(y_hbm.at[idx_v], rows_v, sem).wait()

        def col(v, _):
            s = pl.ds(v * 16, 16)
            for t in range(TC_):
                out_v[t, s] = rows_v[2 * t, s] + rows_v[2 * t + 1, s]
            return _

        lax.fori_loop(0, D // 16, col, 0)
        pltpu.sync_copy(out_v, out_hbm.at[pl.ds(tbase, TC_)])


def _combine(y_pad, pos_flat):
    mesh = plsc.VectorSubcoreMesh(core_axis_name="c", subcore_axis_name="s")
    k = pl.kernel(
        _combine_body,
        out_type=jax.ShapeDtypeStruct((T, D), jnp.float32),
        mesh=mesh,
        scratch_types=[
            pltpu.VMEM((32,), jnp.int32),
            pltpu.VMEM((32, D), jnp.float32),
            pltpu.VMEM((16, D), jnp.float32),
            pltpu.SemaphoreType.DMA,
        ],
    )
    return k(y_pad, pos_flat)


def kernel(hidden_states, W_router, W1, b1, W2, b2):
    b, s, d = hidden_states.shape
    flat = hidden_states.reshape(-1, d)

    a1, a2, g1, g2, colsum = _router(flat, W_router)
    colsum = colsum[0]
    aux_loss = E * jnp.sum((colsum / T) * (colsum / (jnp.sum(colsum) + 1e-8)))

    e_flat = jnp.concatenate([a1, a2], axis=1).reshape(-1)
    g_flat = jnp.concatenate([g1, g2], axis=1).reshape(-1)
    tok_flat = jnp.arange(TK, dtype=jnp.int32) // K
    onehot = (e_flat[:, None] == jnp.arange(E, dtype=jnp.int32)[None, :])
    csum = jnp.cumsum(onehot.astype(jnp.int32), axis=0)
    counts = csum[-1]
    rank = jnp.take_along_axis(csum, e_flat[:, None], axis=1)[:, 0] - 1
    blocks_pe = (counts + M - 1) // M
    blk_cum = jnp.cumsum(blocks_pe)
    pad_start = (blk_cum - blocks_pe) * M
    pad_pos = (pad_start[e_flat] + rank).astype(jnp.int32)
    gate_pad = jnp.zeros((P,), jnp.float32).at[pad_pos].set(g_flat)
    block_expert = jnp.minimum(
        jnp.searchsorted(blk_cum, jnp.arange(NB), side="right"), E - 1
    ).astype(jnp.int32)

    used = blk_cum[-1]
    nsteps = used * NF
    g_arr = jnp.minimum(jnp.arange(NB * NF, dtype=jnp.int32), nsteps - 1)
    jmap = g_arr // NF
    fmap = g_arr % NF
    emap = block_expert[jmap]
    nsteps_a = jnp.full((1,), nsteps, jnp.int32)

    x_pad = _dispatch(flat, tok_flat, pad_pos)
    y_pad = _ffn(jmap, fmap, emap, nsteps_a, x_pad, gate_pad, W1, b1, W2, b2)
    out = _combine(y_pad, pad_pos)

    return out.reshape(b, s, d), aux_loss


# speedup vs baseline: 1.1537x; 1.1537x over previous
import functools

import jax
import jax.numpy as jnp
from jax import lax
from jax.experimental import pallas as pl
from jax.experimental.pallas import tpu as pltpu
from jax.experimental.pallas import tpu_sc as plsc

T, D, E, F = 2048, 2048, 8, 4096
K = 2
TK = T * K
M = 512
NB = TK // M + E
P = NB * M
FB = 1024
NF = F // FB

RB = 512
NC, NS = 2, 16
NW = NC * NS


def _router_body(x_ref, wr_ref, a1_ref, a2_ref, g1_ref, g2_ref, cs_ref):
    r = pl.program_id(0)
    x = x_ref[...]
    logits = jnp.dot(x, wr_ref[...], preferred_element_type=jnp.float32)
    iota = lax.broadcasted_iota(jnp.int32, (RB, E), 1)
    m1 = jnp.max(logits, axis=1, keepdims=True)
    is1 = logits == m1
    a1 = jnp.min(jnp.where(is1, iota, E), axis=1, keepdims=True)
    l2 = jnp.where(iota == a1, -jnp.inf, logits)
    m2 = jnp.max(l2, axis=1, keepdims=True)
    a2 = jnp.min(jnp.where(l2 == m2, iota, E), axis=1, keepdims=True)
    g1 = jax.nn.sigmoid(m1 - m2)
    a1_ref[...] = a1
    a2_ref[...] = a2
    g1_ref[...] = g1
    g2_ref[...] = 1.0 - g1
    z = jnp.exp(logits - m1)
    probs = z / jnp.sum(z, axis=1, keepdims=True)
    part = jnp.sum(probs, axis=0, keepdims=True)

    @pl.when(r == 0)
    def _():
        cs_ref[...] = part

    @pl.when(r != 0)
    def _():
        cs_ref[...] += part


def _router(flat, W_router):
    nr = T // RB
    outs = (
        jax.ShapeDtypeStruct((T, 1), jnp.int32),
        jax.ShapeDtypeStruct((T, 1), jnp.int32),
        jax.ShapeDtypeStruct((T, 1), jnp.float32),
        jax.ShapeDtypeStruct((T, 1), jnp.float32),
        jax.ShapeDtypeStruct((1, E), jnp.float32),
    )
    col = pl.BlockSpec((RB, 1), lambda r: (r, 0))
    return pl.pallas_call(
        _router_body,
        grid=(nr,),
        in_specs=[
            pl.BlockSpec((RB, D), lambda r: (r, 0)),
            pl.BlockSpec((D, E), lambda r: (0, 0)),
        ],
        out_specs=(col, col, col, col, pl.BlockSpec((1, E), lambda r: (0, 0))),
        out_shape=outs,
    )(flat, W_router)


DC = 16
DNCH = TK // NW // DC


def _dispatch_body(src_hbm, tok_hbm, pos_hbm, out_hbm,
                   tok_v, pos_v, bufs, gsem, ssem):
    wid = lax.axis_index("s") * NC + lax.axis_index("c")
    pltpu.sync_copy(tok_hbm.at[wid], tok_v)
    pltpu.sync_copy(pos_hbm.at[wid], pos_v)
    gathers = []
    scatters = []
    gathers.append(pltpu.async_copy(src_hbm.at[tok_v.at[0]], bufs.at[0], gsem))
    for c in range(DNCH):
        gathers[c].wait()
        if c + 1 < DNCH:
            if c >= 1:
                scatters[c - 1].wait()
            gathers.append(pltpu.async_copy(
                src_hbm.at[tok_v.at[c + 1]], bufs.at[(c + 1) % 2], gsem))
        scatters.append(pltpu.async_copy(
            bufs.at[c % 2], out_hbm.at[pos_v.at[c]], ssem))
    scatters[DNCH - 2].wait()
    scatters[DNCH - 1].wait()


def _dispatch(flat, tok_ids, pad_pos):
    mesh = plsc.VectorSubcoreMesh(core_axis_name="c", subcore_axis_name="s")
    k = pl.kernel(
        _dispatch_body,
        out_type=jax.ShapeDtypeStruct((P, D), jnp.float32),
        mesh=mesh,
        scratch_types=[
            pltpu.VMEM((DNCH, DC), jnp.int32),
            pltpu.VMEM((DNCH, DC), jnp.int32),
            pltpu.VMEM((2, DC, D), jnp.float32),
            pltpu.SemaphoreType.DMA,
            pltpu.SemaphoreType.DMA,
        ],
    )
    return k(flat, tok_ids.reshape(NW, DNCH, DC), pad_pos.reshape(NW, DNCH, DC))


def _ffn_body(jm_ref, fm_ref, em_ref, ns_ref,
              x_ref, g_ref, w1_ref, b1_ref, w2_ref, b2_ref, o_ref, xb_scr):
    g_id = pl.program_id(0)
    active = g_id < ns_ref[0]
    f = fm_ref[g_id]

    @pl.when(active)
    def _():
        @pl.when(f == 0)
        def _():
            xb_scr[...] = x_ref[...].astype(jnp.bfloat16)

        xb = xb_scr[...]
        w1 = w1_ref[0].astype(jnp.bfloat16)
        h = jnp.dot(xb, w1, preferred_element_type=jnp.float32)
        h = jax.nn.gelu(h + b1_ref[0])
        w2 = w2_ref[0].astype(jnp.bfloat16)
        y = jnp.dot(h.astype(jnp.bfloat16), w2, preferred_element_type=jnp.float32)

        @pl.when(f == 0)
        def _():
            o_ref[...] = y

        @pl.when((f != 0) & (f != NF - 1))
        def _():
            o_ref[...] += y

        @pl.when(f == NF - 1)
        def _():
            o_ref[...] = (o_ref[...] + y + b2_ref[0]) * g_ref[0]


def _ffn(jmap, fmap, emap, nsteps, x_pad, gate_pad, W1, b1, W2, b2):
    gate3 = gate_pad.reshape(NB, M, 1)
    b1r = b1.reshape(E, 1, F)
    b2r = b2.reshape(E, 1, D)
    grid_spec = pltpu.PrefetchScalarGridSpec(
        num_scalar_prefetch=4,
        grid=(NB * NF,),
        in_specs=[
            pl.BlockSpec((M, D), lambda g, jm, fm, em, ns: (jm[g], 0)),
            pl.BlockSpec((1, M, 1), lambda g, jm, fm, em, ns: (jm[g], 0, 0)),
            pl.BlockSpec((1, D, FB), lambda g, jm, fm, em, ns: (em[g], 0, fm[g])),
            pl.BlockSpec((1, 1, FB), lambda g, jm, fm, em, ns: (em[g], 0, fm[g])),
            pl.BlockSpec((1, FB, D), lambda g, jm, fm, em, ns: (em[g], fm[g], 0)),
            pl.BlockSpec((1, 1, D), lambda g, jm, fm, em, ns: (em[g], 0, 0)),
        ],
        out_specs=pl.BlockSpec((M, D), lambda g, jm, fm, em, ns: (jm[g], 0)),
        scratch_shapes=[pltpu.VMEM((M, D), jnp.bfloat16)],
    )
    return pl.pallas_call(
        _ffn_body,
        grid_spec=grid_spec,
        out_shape=jax.ShapeDtypeStruct((P, D), jnp.float32),
    )(jmap, fmap, emap, nsteps, x_pad, gate3, W1, b1r, W2, b2r)


def _combine_body(y_hbm, pos_hbm, out_hbm, idx_v, rows_v, out_v, sem):
    wid = lax.axis_index("s") * NC + lax.axis_index("c")
    tok_per_w = T // NW
    TC_ = 16
    for c in range(tok_per_w // TC_):
        tbase = wid * tok_per_w + c * TC_
        pltpu.sync_copy(pos_hbm.at[pl.ds(tbase * K, TC_ * K)], idx_v)
        pltpu.async_copy(y_hbm.at[idx_v], rows_v, sem).wait()

        def col(v, _):
            s = pl.ds(v * 16, 16)
            for t in range(TC_):
                out_v[t, s] = rows_v[2 * t, s] + rows_v[2 * t + 1, s]
            return _

        lax.fori_loop(0, D // 16, col, 0)
        pltpu.sync_copy(out_v, out_hbm.at[pl.ds(tbase, TC_)])


def _combine(y_pad, pos_flat):
    mesh = plsc.VectorSubcoreMesh(core_axis_name="c", subcore_axis_name="s")
    k = pl.kernel(
        _combine_body,
        out_type=jax.ShapeDtypeStruct((T, D), jnp.float32),
        mesh=mesh,
        scratch_types=[
            pltpu.VMEM((32,), jnp.int32),
            pltpu.VMEM((32, D), jnp.float32),
            pltpu.VMEM((16, D), jnp.float32),
            pltpu.SemaphoreType.DMA,
        ],
    )
    return k(y_pad, pos_flat)


def kernel(hidden_states, W_router, W1, b1, W2, b2):
    b, s, d = hidden_states.shape
    flat = hidden_states.reshape(-1, d)

    a1, a2, g1, g2, colsum = _router(flat, W_router)
    colsum = colsum[0]
    aux_loss = E * jnp.sum((colsum / T) * (colsum / (jnp.sum(colsum) + 1e-8)))

    e_flat = jnp.concatenate([a1, a2], axis=1).reshape(-1)
    g_flat = jnp.concatenate([g1, g2], axis=1).reshape(-1)
    tok_flat = jnp.arange(TK, dtype=jnp.int32) // K
    onehot = (e_flat[:, None] == jnp.arange(E, dtype=jnp.int32)[None, :])
    csum = jnp.cumsum(onehot.astype(jnp.int32), axis=0)
    counts = csum[-1]
    rank = jnp.take_along_axis(csum, e_flat[:, None], axis=1)[:, 0] - 1
    blocks_pe = (counts + M - 1) // M
    blk_cum = jnp.cumsum(blocks_pe)
    pad_start = (blk_cum - blocks_pe) * M
    pad_pos = (pad_start[e_flat] + rank).astype(jnp.int32)
    gate_pad = jnp.zeros((P,), jnp.float32).at[pad_pos].set(g_flat)
    block_expert = jnp.minimum(
        jnp.searchsorted(blk_cum, jnp.arange(NB), side="right"), E - 1
    ).astype(jnp.int32)

    used = blk_cum[-1]
    nsteps = used * NF
    g_arr = jnp.minimum(jnp.arange(NB * NF, dtype=jnp.int32), nsteps - 1)
    jmap = g_arr // NF
    fmap = g_arr % NF
    emap = block_expert[jmap]
    nsteps_a = jnp.full((1,), nsteps, jnp.int32)

    x_pad = _dispatch(flat, tok_flat, pad_pos)
    y_pad = _ffn(jmap, fmap, emap, nsteps_a, x_pad, gate_pad, W1, b1, W2, b2)
    out = _combine(y_pad, pad_pos)

    return out.reshape(b, s, d), aux_loss
